# baseline (speedup 1.0000x reference)
import functools

import jax
import jax.numpy as jnp
from jax import lax
from jax.experimental import pallas as pl
from jax.experimental.pallas import tpu as pltpu
from jax.experimental.pallas import tpu_sc as plsc

VOCAB = 1000000
EMBED = 64
B_ROWS = 4096
B_COLS = 50
TOTAL = B_ROWS * B_COLS
CHUNK = 128
N_CHUNK_ROWS = TOTAL // CHUNK

_info = plsc.get_sparse_core_info()
NC, NS = _info.num_cores, _info.num_subcores
NW = NC * NS
ROWS_PER_W = N_CHUNK_ROWS // NW
NBUF = 5
LANES = 16


def _make_kernel():
    mesh = plsc.VectorSubcoreMesh(core_axis_name="c", subcore_axis_name="s")

    @functools.partial(
        pl.kernel,
        mesh=mesh,
        out_type=jax.ShapeDtypeStruct((TOTAL, 2 * EMBED), jnp.float32),
        scratch_types=[
            pltpu.VMEM((ROWS_PER_W, CHUNK), jnp.int32),
            pltpu.VMEM((ROWS_PER_W, CHUNK), jnp.int32),
            pltpu.VMEM((NBUF, CHUNK, 2 * EMBED), jnp.float32),
            [pltpu.SemaphoreType.DMA] * NBUF,
        ],
    )
    def k(idx_hbm, table_hbm, out_hbm, idx_v, gidx_v, pair_v, sems):
        wid = lax.axis_index("s") * NC + lax.axis_index("c")
        out_base = wid * ROWS_PER_W * CHUNK

        pltpu.sync_copy(idx_hbm.at[wid], idx_v)

        for j in range(ROWS_PER_W):
            for v in range(CHUNK // LANES):
                sl = pl.ds(v * LANES, LANES)
                gidx_v[j, sl] = lax.shift_right_logical(idx_v[j, sl], 1)

        for b in range(NBUF):
            pltpu.async_copy(table_hbm.at[gidx_v.at[b]], pair_v.at[b], sems[b])

        @pl.loop(0, ROWS_PER_W, step=NBUF)
        def _ring(g0):
            for b in range(NBUF):
                g = g0 + b
                pltpu.make_async_copy(table_hbm.at[gidx_v.at[g]], pair_v.at[b],
                                      sems[b]).wait()
                off = pl.multiple_of(out_base + g * CHUNK, CHUNK)
                pltpu.sync_copy(pair_v.at[b], out_hbm.at[pl.ds(off, CHUNK)])
                nxt = g + NBUF

                @pl.when(nxt < ROWS_PER_W)
                def _():
                    pltpu.async_copy(table_hbm.at[gidx_v.at[nxt]], pair_v.at[b],
                                     sems[b])

    return k


_kernel_call = _make_kernel()


def kernel(inputs, embeddings):
    idx = jnp.reshape(inputs.astype(jnp.int32), (NW, ROWS_PER_W, CHUNK))
    table2 = jnp.reshape(embeddings, (VOCAB // 2, 2 * EMBED))
    pairs = _kernel_call(idx, table2)
    half = jnp.reshape(inputs.astype(jnp.int32) & 1, (TOTAL, 1))
    out = jnp.where(half == 0, pairs[:, :EMBED], pairs[:, EMBED:])
    return jnp.reshape(out, (B_ROWS, B_COLS, EMBED))


# speedup vs baseline: 1.0320x; 1.0320x over previous
import functools

import jax
import jax.numpy as jnp
from jax import lax
from jax.experimental import pallas as pl
from jax.experimental.pallas import tpu as pltpu
from jax.experimental.pallas import tpu_sc as plsc

VOCAB = 1000000
EMBED = 64
B_ROWS = 4096
B_COLS = 50
CHUNK = 128
TOTAL = B_ROWS * B_COLS

_info = plsc.get_sparse_core_info()
NC, NS = _info.num_cores, _info.num_subcores
NW = NC * NS
NBUF = 2
LANES = 16


def _make_kernel():
    mesh = plsc.VectorSubcoreMesh(core_axis_name="c", subcore_axis_name="s")

    @functools.partial(
        pl.kernel,
        mesh=mesh,
        compiler_params=pltpu.CompilerParams(needs_layout_passes=False),
        out_type=jax.ShapeDtypeStruct((B_COLS, EMBED, B_ROWS), jnp.float32),
        scratch_types=[
            pltpu.VMEM((B_COLS, CHUNK), jnp.int32),
            pltpu.VMEM((B_COLS, CHUNK), jnp.int32),
            pltpu.VMEM((B_COLS, CHUNK), jnp.int32),
            pltpu.VMEM((NBUF, CHUNK, 2 * EMBED), jnp.float32),
            pltpu.VMEM((EMBED, CHUNK), jnp.float32),
            [pltpu.SemaphoreType.DMA] * NBUF,
        ],
    )
    def k(idx_hbm, table_hbm, out_hbm, idx_v, gidx_v, hm_v, pair_v,
          st_v, sems):
        wid = lax.axis_index("s") * NC + lax.axis_index("c")
        b0 = pl.multiple_of(wid * CHUNK, CHUNK)

        pltpu.sync_copy(idx_hbm.at[:, pl.ds(b0, CHUNK)], idx_v)

        for j in range(B_COLS):
            for g in range(CHUNK // LANES):
                sl = pl.ds(g * LANES, LANES)
                v = idx_v[j, sl]
                gidx_v[j, sl] = lax.shift_right_logical(v, 1)
                hm_v[j, sl] = (v & 1) * EMBED

        for b in range(NBUF):
            pltpu.async_copy(table_hbm.at[gidx_v.at[b]], pair_v.at[b], sems[b])

        iotas = [lax.iota(jnp.int32, LANES) + g * LANES
                 for g in range(CHUNK // LANES)]

        @pl.loop(0, B_COLS, step=NBUF)
        def _ring(s0):
            for b in range(NBUF):
                s = s0 + b
                pltpu.make_async_copy(table_hbm.at[gidx_v.at[s]], pair_v.at[b],
                                      sems[b]).wait()
                hvs = [hm_v[s, pl.ds(g * LANES, LANES)]
                       for g in range(CHUNK // LANES)]
                for g in range(CHUNK // LANES):
                    rows = iotas[g]
                    cols0 = hvs[g]
                    for e in range(EMBED):
                        vals = plsc.load_gather(pair_v.at[b], [rows, cols0 + e])
                        st_v[e, pl.ds(g * LANES, LANES)] = vals
                pltpu.sync_copy(st_v, out_hbm.at[s, :, pl.ds(b0, CHUNK)])
                nxt = s + NBUF

                @pl.when(nxt < B_COLS)
                def _():
                    pltpu.async_copy(table_hbm.at[gidx_v.at[nxt]], pair_v.at[b],
                                     sems[b])

    return k


_kernel_call = _make_kernel()


def kernel(inputs, embeddings):
    idx_t = jnp.transpose(inputs.astype(jnp.int32))
    table2 = jnp.reshape(embeddings, (VOCAB // 2, 2 * EMBED))
    out3 = _kernel_call(idx_t, table2)
    return jnp.transpose(out3, (2, 0, 1))


# speedup vs baseline: 1.2564x; 1.2175x over previous
import functools

import jax
import jax.numpy as jnp
from jax import lax
from jax.experimental import pallas as pl
from jax.experimental.pallas import tpu as pltpu
from jax.experimental.pallas import tpu_sc as plsc

VOCAB = 1000000
EMBED = 64
B_ROWS = 4096
B_COLS = 50
TOTAL = B_ROWS * B_COLS
CHUNK = 128
N_CHUNK_ROWS = TOTAL // CHUNK

_info = plsc.get_sparse_core_info()
NC, NS = _info.num_cores, _info.num_subcores
NW = NC * NS
ROWS_PER_W = N_CHUNK_ROWS // NW
NBUF = 10


def _make_kernel():
    mesh = plsc.VectorSubcoreMesh(core_axis_name="c", subcore_axis_name="s")

    @functools.partial(
        pl.kernel,
        mesh=mesh,
        compiler_params=pltpu.CompilerParams(use_tc_tiling_on_sc=False),
        out_type=jax.ShapeDtypeStruct((TOTAL, EMBED), jnp.float32),
        scratch_types=[
            pltpu.VMEM((ROWS_PER_W, CHUNK), jnp.int32),
            pltpu.VMEM((NBUF, CHUNK, EMBED), jnp.float32),
            [pltpu.SemaphoreType.DMA] * NBUF,
        ],
    )
    def k(idx_hbm, table_hbm, out_hbm, idx_v, rows_v, sems):
        wid = lax.axis_index("s") * NC + lax.axis_index("c")
        out_base = wid * ROWS_PER_W * CHUNK

        pltpu.sync_copy(idx_hbm.at[wid], idx_v)

        for b in range(NBUF):
            pltpu.async_copy(table_hbm.at[idx_v.at[b]], rows_v.at[b], sems[b])

        @pl.loop(0, ROWS_PER_W, step=NBUF)
        def _ring(g0):
            for b in range(NBUF):
                g = g0 + b
                pltpu.make_async_copy(table_hbm.at[idx_v.at[g]], rows_v.at[b],
                                      sems[b]).wait()
                off = pl.multiple_of(out_base + g * CHUNK, CHUNK)
                pltpu.sync_copy(rows_v.at[b], out_hbm.at[pl.ds(off, CHUNK)])
                nxt = g + NBUF

                @pl.when(nxt < ROWS_PER_W)
                def _():
                    pltpu.async_copy(table_hbm.at[idx_v.at[nxt]], rows_v.at[b],
                                     sems[b])

    return k


_kernel_call = _make_kernel()


def kernel(inputs, embeddings):
    idx = jnp.reshape(inputs.astype(jnp.int32), (NW, ROWS_PER_W, CHUNK))
    out = _kernel_call(idx, embeddings)
    return jnp.reshape(out, (B_ROWS, B_COLS, EMBED))


# speedup vs baseline: 1.2765x; 1.0159x over previous
import functools

import jax
import jax.numpy as jnp
from jax import lax
from jax.experimental import pallas as pl
from jax.experimental.pallas import tpu as pltpu
from jax.experimental.pallas import tpu_sc as plsc

VOCAB = 1000000
EMBED = 64
B_ROWS = 4096
B_COLS = 50
CHUNK = 128

_info = plsc.get_sparse_core_info()
NC, NS = _info.num_cores, _info.num_subcores
NW = NC * NS
NBUF = 10


def _make_kernel():
    mesh = plsc.VectorSubcoreMesh(core_axis_name="c", subcore_axis_name="s")

    @functools.partial(
        pl.kernel,
        mesh=mesh,
        compiler_params=pltpu.CompilerParams(use_tc_tiling_on_sc=False),
        out_type=jax.ShapeDtypeStruct((B_COLS, B_ROWS, EMBED), jnp.float32),
        scratch_types=[
            pltpu.VMEM((B_COLS, CHUNK), jnp.int32),
            pltpu.VMEM((NBUF, CHUNK, EMBED), jnp.float32),
            [pltpu.SemaphoreType.DMA] * NBUF,
        ],
    )
    def k(idx_hbm, table_hbm, out_hbm, idx_v, rows_v, sems):
        wid = lax.axis_index("s") * NC + lax.axis_index("c")
        b0 = pl.multiple_of(wid * CHUNK, CHUNK)

        pltpu.sync_copy(idx_hbm.at[:, pl.ds(b0, CHUNK)], idx_v)

        for b in range(NBUF):
            pltpu.async_copy(table_hbm.at[idx_v.at[b]], rows_v.at[b], sems[b])

        @pl.loop(0, B_COLS, step=NBUF)
        def _ring(s0):
            for b in range(NBUF):
                s = s0 + b
                pltpu.make_async_copy(table_hbm.at[idx_v.at[s]], rows_v.at[b],
                                      sems[b]).wait()
                pltpu.sync_copy(rows_v.at[b],
                                out_hbm.at[s].at[pl.ds(b0, CHUNK)])
                nxt = s + NBUF

                @pl.when(nxt < B_COLS)
                def _():
                    pltpu.async_copy(table_hbm.at[idx_v.at[nxt]], rows_v.at[b],
                                     sems[b])

    return k


_kernel_call = _make_kernel()


def kernel(inputs, embeddings):
    idx_t = jnp.transpose(inputs.astype(jnp.int32))
    out_d = _kernel_call(idx_t, embeddings)
    return jnp.transpose(out_d, (1, 0, 2))
